# Initial kernel scaffold; baseline (speedup 1.0000x reference)
import jax
import jax.numpy as jnp
from jax.experimental import pallas as pl


def kernel(x, w_gate, W1, b1, W2, b2):
    raise NotImplementedError("write your pallas kernel here")


# speedup vs baseline: 5.7166x; 5.7166x over previous
import functools

import jax
import jax.numpy as jnp
from jax.experimental import pallas as pl
from jax.experimental.pallas import tpu as pltpu

B, D, L = 4, 1024, 8192
E, K, OC = 8, 2, 10
LP = L - 2
DC = 256
ND = D // DC
MP = 16


def _kernel(x_ref, xg_ref, wall_ref, wg_ref, w2_ref, b1_ref, b2_ref, out_ref,
            z_ref):
    nd = pl.program_id(1)

    @pl.when(nd == 0)
    def _init():
        z_ref[...] = jnp.zeros_like(z_ref)

    xb = x_ref[0]
    z_ref[...] += jnp.dot(wall_ref[...], xb, preferred_element_type=jnp.float32)

    @pl.when(nd == ND - 1)
    def _finalize():
        logits = jnp.dot(xg_ref[0], wg_ref[...],
                         preferred_element_type=jnp.float32)
        sm = jax.nn.softmax(logits, axis=-1)
        iota = jax.lax.broadcasted_iota(jnp.int32, (1, E), 1)
        v1 = jnp.max(sm)
        i1 = jnp.argmax(sm[0, :])
        masked = jnp.where(iota == i1, -jnp.inf, sm)
        v2 = jnp.max(masked)
        i2 = jnp.argmax(masked[0, :])
        denom = v1 + v2 + 1e-6
        gates = jnp.where(iota == i1, v1 / denom,
                          jnp.where(iota == i2, v2 / denom, 0.0))
        w_eff = jnp.sum(w2_ref[...] * gates[:, :, None], axis=1)
        b_eff = jnp.sum(b2_ref[...] * gates, axis=1, keepdims=True)
        z = z_ref[...]
        y = (z[0:MP, 0:LP] + z[MP:2 * MP, 1:LP + 1]
             + z[2 * MP:3 * MP, 2:LP + 2])
        h = jnp.tanh(y + b1_ref[...])
        out_ref[0] = (jnp.dot(w_eff, h, preferred_element_type=jnp.float32)
                      + b_eff)


@jax.jit
def kernel(x, w_gate, W1, b1, W2, b2):
    w_t = jnp.transpose(W1, (2, 0, 1))
    w_all = jnp.pad(w_t, ((0, 0), (0, MP - OC), (0, 0))).reshape(3 * MP, D)
    w2r = W2[:, :, 0].reshape(OC, E, OC)
    w2r = jnp.pad(w2r, ((0, 0), (0, 0), (0, MP - OC)))
    b2r = b2.reshape(OC, E)
    b1p = jnp.pad(b1, (0, MP - OC)).reshape(MP, 1)
    xgr = x[:, :, L - 6:L - 1].reshape(B, 1, D * 5)

    out = pl.pallas_call(
        _kernel,
        grid=(B, ND),
        in_specs=[
            pl.BlockSpec((1, DC, L), lambda b, nd: (b, nd, 0)),
            pl.BlockSpec((1, 1, D * 5), lambda b, nd: (b, 0, 0)),
            pl.BlockSpec((3 * MP, DC), lambda b, nd: (0, nd)),
            pl.BlockSpec((D * 5, E), lambda b, nd: (0, 0)),
            pl.BlockSpec((OC, E, MP), lambda b, nd: (0, 0, 0)),
            pl.BlockSpec((MP, 1), lambda b, nd: (0, 0)),
            pl.BlockSpec((OC, E), lambda b, nd: (0, 0)),
        ],
        out_specs=pl.BlockSpec((1, OC, LP), lambda b, nd: (b, 0, 0)),
        out_shape=jax.ShapeDtypeStruct((B, OC, LP), jnp.float32),
        scratch_shapes=[
            pltpu.VMEM((3 * MP, L), jnp.float32),
        ],
        compiler_params=pltpu.CompilerParams(
            dimension_semantics=("parallel", "arbitrary"),
        ),
    )(x, xgr, w_all, w_gate, w2r, b1p, b2r)
    return out


# speedup vs baseline: 5.7173x; 1.0001x over previous
import functools

import jax
import jax.numpy as jnp
from jax.experimental import pallas as pl
from jax.experimental.pallas import tpu as pltpu

B, D, L = 4, 1024, 8192
E, K, OC = 8, 2, 10
LP = L - 2
DC = 256
ND = D // DC
MP = 16


def _kernel(x_ref, xg_ref, wall_ref, wg_ref, w2_ref, b1_ref, b2_ref, out_ref,
            z_ref):
    nd = pl.program_id(1)

    @pl.when(nd == 0)
    def _init():
        z_ref[...] = jnp.zeros_like(z_ref)

    xb = x_ref[0].astype(jnp.bfloat16)
    z_ref[...] += jnp.dot(wall_ref[...], xb, preferred_element_type=jnp.float32)

    @pl.when(nd == ND - 1)
    def _finalize():
        logits = jnp.dot(xg_ref[0], wg_ref[...],
                         preferred_element_type=jnp.float32)
        sm = jax.nn.softmax(logits, axis=-1)
        iota = jax.lax.broadcasted_iota(jnp.int32, (1, E), 1)
        v1 = jnp.max(sm)
        i1 = jnp.argmax(sm[0, :])
        masked = jnp.where(iota == i1, -jnp.inf, sm)
        v2 = jnp.max(masked)
        i2 = jnp.argmax(masked[0, :])
        denom = v1 + v2 + 1e-6
        gates = jnp.where(iota == i1, v1 / denom,
                          jnp.where(iota == i2, v2 / denom, 0.0))
        w_eff = jnp.sum(w2_ref[...] * gates[:, :, None], axis=1)
        b_eff = jnp.sum(b2_ref[...] * gates, axis=1, keepdims=True)
        z = z_ref[...]
        y = (z[0:MP, 0:LP] + z[MP:2 * MP, 1:LP + 1]
             + z[2 * MP:3 * MP, 2:LP + 2])
        h = jnp.tanh(y + b1_ref[...])
        out_ref[0] = (jnp.dot(w_eff, h, preferred_element_type=jnp.float32)
                      + b_eff)


@jax.jit
def kernel(x, w_gate, W1, b1, W2, b2):
    w_t = jnp.transpose(W1, (2, 0, 1))
    w_all = jnp.pad(w_t, ((0, 0), (0, MP - OC), (0, 0))).reshape(3 * MP, D)
    w_all = w_all.astype(jnp.bfloat16)
    w2r = W2[:, :, 0].reshape(OC, E, OC)
    w2r = jnp.pad(w2r, ((0, 0), (0, 0), (0, MP - OC)))
    b2r = b2.reshape(OC, E)
    b1p = jnp.pad(b1, (0, MP - OC)).reshape(MP, 1)
    xgr = x[:, :, L - 6:L - 1].reshape(B, 1, D * 5)

    out = pl.pallas_call(
        _kernel,
        grid=(B, ND),
        in_specs=[
            pl.BlockSpec((1, DC, L), lambda b, nd: (b, nd, 0)),
            pl.BlockSpec((1, 1, D * 5), lambda b, nd: (b, 0, 0)),
            pl.BlockSpec((3 * MP, DC), lambda b, nd: (0, nd)),
            pl.BlockSpec((D * 5, E), lambda b, nd: (0, 0)),
            pl.BlockSpec((OC, E, MP), lambda b, nd: (0, 0, 0)),
            pl.BlockSpec((MP, 1), lambda b, nd: (0, 0)),
            pl.BlockSpec((OC, E), lambda b, nd: (0, 0)),
        ],
        out_specs=pl.BlockSpec((1, OC, LP), lambda b, nd: (b, 0, 0)),
        out_shape=jax.ShapeDtypeStruct((B, OC, LP), jnp.float32),
        scratch_shapes=[
            pltpu.VMEM((3 * MP, L), jnp.float32),
        ],
        compiler_params=pltpu.CompilerParams(
            dimension_semantics=("parallel", "arbitrary"),
        ),
    )(x, xgr, w_all, w_gate, w2r, b1p, b2r)
    return out
